# baseline (speedup 1.0000x reference)
import functools

import numpy as np
import jax
import jax.numpy as jnp
from jax import lax
from jax.experimental import pallas as pl
from jax.experimental.pallas import tpu as pltpu
from jax.experimental.pallas import tpu_sc as plsc

NC = 2
NS = 16
LN = 16
NW = NC * NS
CHUNK = 512
SUB = CHUNK // LN
EPS = np.float32(1e-6)

_PIO2_HI = np.float32(1.5707962513e+00)
_PIO2_LO = np.float32(7.5497894159e-08)
_PS0 = np.float32(1.6666586697e-01)
_PS1 = np.float32(-4.2743422091e-02)
_PS2 = np.float32(-8.6563630030e-03)
_QS1 = np.float32(-7.0662963390e-01)
_MAGIC = np.int32(0x5F3759DF)


def _sqrtf(z):
    i = lax.bitcast_convert_type(z, jnp.int32)
    y = lax.bitcast_convert_type(_MAGIC - lax.shift_right_logical(i, 1),
                                 jnp.float32)
    hz = jnp.float32(0.5) * z
    y = y * (jnp.float32(1.5) - hz * y * y)
    y = y * (jnp.float32(1.5) - hz * y * y)
    s = z * y
    return jnp.float32(0.5) * (s + z / s)


def _acosf(x):
    ax = jnp.abs(x)
    half = jnp.float32(0.5)
    one = jnp.float32(1.0)
    lo = ax < half
    neg = x < jnp.float32(0.0)
    z = jnp.where(lo, x * x, half * jnp.where(neg, one + x, one - x))
    p = z * (_PS0 + z * (_PS1 + z * _PS2))
    q = one + z * _QS1
    r = p / q
    s = _sqrtf(z)
    r1 = _PIO2_HI - (x - (_PIO2_LO - x * r))
    r2 = jnp.float32(2.0) * (_PIO2_HI - (s + (r * s - _PIO2_LO)))
    df = lax.bitcast_convert_type(
        lax.bitcast_convert_type(s, jnp.int32) & jnp.int32(~0xFFF),
        jnp.float32)
    c = (z - df * df) / (s + df)
    r3 = jnp.float32(2.0) * (df + (r * s + c))
    return jnp.where(lo, r1, jnp.where(neg, r2, r3))


def _make_sc_call(Bn, L, nseg, ncut):
    slab = (L // NW) * L
    shift = L.bit_length() - 1

    mesh = plsc.VectorSubcoreMesh(core_axis_name="c", subcore_axis_name="s",
                                  num_cores=NC, num_subcores=NS)

    @functools.partial(
        pl.kernel,
        out_type=jax.ShapeDtypeStruct((NW * LN,), jnp.float32),
        mesh=mesh,
        compiler_params=pltpu.CompilerParams(needs_layout_passes=False,
                                             use_tc_tiling_on_sc=False),
        scratch_types=[
            pltpu.VMEM((slab,), jnp.uint8),
            pltpu.VMEM((slab + 3 * CHUNK,), jnp.int32),
            pltpu.VMEM((6 * Bn * L,), jnp.float32),
            pltpu.VMEM((ncut * LN,), jnp.float32),
            pltpu.VMEM((4 * CHUNK,), jnp.int32),
            pltpu.VMEM((4 * CHUNK,), jnp.int32),
            pltpu.VMEM((CHUNK,), jnp.float32),
            pltpu.VMEM((CHUNK,), jnp.float32),
            pltpu.VMEM((CHUNK,), jnp.float32),
            pltpu.VMEM((CHUNK,), jnp.float32),
            pltpu.VMEM((4 * CHUNK,), jnp.float32),
            pltpu.VMEM((4 * CHUNK,), jnp.float32),
            pltpu.VMEM((LN,), jnp.float32),
            pltpu.SemaphoreType.DMA,
            pltpu.SemaphoreType.DMA,
        ],
    )
    def sc_call(mask_hbm, geom_hbm, cut_hbm, coeff_hbm, out_hbm,
                mslab, pairs, geomv, cutv, offs0, offs1,
                xls0, xls1, wts0, wts1, cbuf0, cbuf1, accv, sem0, sem1):
        bufs = ((offs0, xls0, wts0, cbuf0, sem0),
                (offs1, xls1, wts1, cbuf1, sem1))
        wid = lax.axis_index("s") * NC + lax.axis_index("c")
        base_pair = wid * slab

        pltpu.sync_copy(mask_hbm.at[pl.ds(base_pair, slab)], mslab)
        pltpu.sync_copy(geom_hbm, geomv)
        pltpu.sync_copy(cut_hbm, cutv)
        accv[...] = jnp.zeros((LN,), jnp.float32)

        iota = lax.iota(jnp.int32, LN)

        pairs_full = pairs.at[pl.ds(0, slab + 3 * CHUNK)]

        def compact_body(q, cnt):
            v8 = mslab[pl.ds(q * (4 * LN), 4 * LN)]
            x = plsc.bitcast(v8, jnp.int32)
            clane = lax.shift_right_logical(x * jnp.int32(0x01010101), 24)
            incl = plsc.cumsum(clane)
            base = cnt + incl - clane
            b0 = x & jnp.int32(0xFF)
            b1 = lax.shift_right_logical(x, 8) & jnp.int32(0xFF)
            b2 = lax.shift_right_logical(x, 16) & jnp.int32(0xFF)
            p1 = b0
            p2 = p1 + b1
            p3 = p2 + b2
            idbase = (base_pair + q * (4 * LN)) + iota * jnp.int32(4)
            for k, pk in ((0, None), (1, p1), (2, p2), (3, p3)):
                bk = lax.shift_right_logical(x, 8 * k) & jnp.int32(0xFF)
                mk = bk != 0
                posk = base if pk is None else base + pk
                plsc.store_scatter(pairs_full, [posk],
                                   idbase + jnp.int32(k), mask=mk)
            return cnt + incl[LN - 1]

        cnt = lax.fori_loop(0, slab // (4 * LN), compact_body, jnp.int32(0))

        nch2 = lax.shift_right_logical(cnt + jnp.int32(2 * CHUNK - 1),
                                       (2 * CHUNK).bit_length() - 1)

        def make_stage(b):
            rb = 6 * b * L

            def stage(g, bi):
                offs, xls, wts, cbuf, sem = bufs[bi]
                cbase = g * CHUNK

                def stage_sub(t, carry):
                    k0 = cbase + t * LN
                    vm = (k0 + iota) < cnt
                    p = jnp.where(vm, pairs[pl.ds(k0, LN)], 0)
                    i = lax.shift_right_logical(p, shift)
                    j = p & jnp.int32(L - 1)

                    def ld(row, idxv):
                        return plsc.load_gather(geomv,
                                                [idxv + jnp.int32(rb + row * L)])

                    cbxi = ld(3, i)
                    cbyi = ld(4, i)
                    cbzi = ld(5, i)
                    xx = ld(0, i) - cbxi
                    xy = ld(1, i) - cbyi
                    xz = ld(2, i) - cbzi
                    yx = ld(3, j) - cbxi
                    yy = ld(4, j) - cbyi
                    yz = ld(5, j) - cbzi
                    nx2 = xx * xx + xy * xy + xz * xz
                    ny2 = yx * yx + yy * yy + yz * yz
                    dot = xx * yx + xy * yy + xz * yz
                    nx = _sqrtf(nx2)
                    ny = _sqrtf(ny2)
                    m1 = (nx > EPS) & (ny > EPS)
                    denom = jnp.where(m1, nx * ny, jnp.float32(1.0))
                    cth = dot / denom
                    good = m1 & ((jnp.float32(1.0) - cth * cth) > EPS) & vm
                    phi = _acosf(jnp.where(good, cth, jnp.float32(0.0)))

                    nlt = jnp.zeros((LN,), jnp.int32)
                    cutsel = cutv[pl.ds(0, LN)]
                    for k in range(ncut):
                        ck = cutv[pl.ds(k * LN, LN)]
                        m = phi > ck
                        nlt = nlt + jnp.where(m, jnp.int32(1), jnp.int32(0))
                        if 1 <= k <= nseg - 1:
                            cutsel = jnp.where(m, ck, cutsel)
                    idx = jnp.clip(nlt - jnp.int32(1), jnp.int32(0),
                                   jnp.int32(nseg - 1))
                    base = ((i * jnp.int32(nseg) + idx) * jnp.int32(4 * L)
                            + lax.shift_left(lax.shift_right_logical(p, 7)
                                             & jnp.int32((L >> 7) - 1), 9)
                            + (p & jnp.int32(127)))
                    base = jnp.where(good, base, 0)

                    offs[pl.ds(t * LN, LN)] = base
                    offs[pl.ds(CHUNK + t * LN, LN)] = base + jnp.int32(128)
                    offs[pl.ds(2 * CHUNK + t * LN, LN)] = base + jnp.int32(256)
                    offs[pl.ds(3 * CHUNK + t * LN, LN)] = base + jnp.int32(384)
                    xls[pl.ds(t * LN, LN)] = phi - cutsel
                    wts[pl.ds(t * LN, LN)] = jnp.where(good, jnp.float32(1.0),
                                                       jnp.float32(0.0))
                    return carry

                lax.fori_loop(0, SUB, stage_sub, jnp.int32(0))
                pltpu.async_copy(coeff_hbm.at[offs], cbuf, sem)

            return stage

        def drain_eval(bi):
            offs, xls, wts, cbuf, sem = bufs[bi]
            pltpu.make_async_copy(coeff_hbm.at[offs], cbuf, sem).wait()

            def eval_sub(t, carry):
                c0 = cbuf[pl.ds(t * LN, LN)]
                c1 = cbuf[pl.ds(CHUNK + t * LN, LN)]
                c2 = cbuf[pl.ds(2 * CHUNK + t * LN, LN)]
                c3 = cbuf[pl.ds(3 * CHUNK + t * LN, LN)]
                xl = xls[pl.ds(t * LN, LN)]
                w = wts[pl.ds(t * LN, LN)]
                tx = xl * xl
                ret = (c3 + c2 * xl) + c1 * tx
                ret = ret + c0 * (tx * xl)
                plsc.addupdate(accv.at[pl.ds(0, LN)], w * ret)
                return carry

            lax.fori_loop(0, SUB, eval_sub, jnp.int32(0))

        for b in range(Bn):
            stage = make_stage(b)
            stage(jnp.int32(0), 0)

            def pair_body(g2, carry):
                stage(g2 * 2 + 1, 1)
                drain_eval(0)
                stage(g2 * 2 + 2, 0)
                drain_eval(1)
                return carry

            lax.fori_loop(jnp.int32(0), nch2, pair_body, jnp.int32(0))
            drain_eval(0)

        pltpu.sync_copy(accv, out_hbm.at[pl.ds(wid * LN, LN)])

    return sc_call


def kernel(CA, CB, coeff, cutoffs, mask):
    Bn, L, _ = CA.shape
    nseg = coeff.shape[2]
    ncut = cutoffs.shape[0]
    geom = jnp.concatenate(
        [jnp.swapaxes(CA, 1, 2), jnp.swapaxes(CB, 1, 2)], axis=1)
    geom = geom.reshape(-1)
    mask_i = mask.reshape(-1).astype(jnp.uint8)
    coeff_flat = jnp.transpose(
        coeff.reshape(L, L // 128, 128, nseg, 4),
        (0, 3, 1, 4, 2)).reshape(-1)
    cut_bro = jnp.broadcast_to(cutoffs[:, None], (ncut, LN)).reshape(-1)
    sc_call = _make_sc_call(Bn, L, nseg, ncut)
    out = sc_call(mask_i, geom, cut_bro, coeff_flat)
    return jnp.sum(out)


# speedup vs baseline: 3.0887x; 3.0887x over previous
import functools

import numpy as np
import jax
import jax.numpy as jnp
from jax import lax
from jax.experimental import pallas as pl
from jax.experimental.pallas import tpu as pltpu
from jax.experimental.pallas import tpu_sc as plsc

NC = 2
NS = 16
LN = 16
NW = NC * NS
CHUNK = 128
SUB = CHUNK // LN
EPS = np.float32(1e-6)

_PIO2_HI = np.float32(1.5707962513e+00)
_PIO2_LO = np.float32(7.5497894159e-08)
_PS0 = np.float32(1.6666586697e-01)
_PS1 = np.float32(-4.2743422091e-02)
_PS2 = np.float32(-8.6563630030e-03)
_QS1 = np.float32(-7.0662963390e-01)
_MAGIC = np.int32(0x5F3759DF)


def _sqrtf(z):
    i = lax.bitcast_convert_type(z, jnp.int32)
    y = lax.bitcast_convert_type(_MAGIC - lax.shift_right_logical(i, 1),
                                 jnp.float32)
    hz = jnp.float32(0.5) * z
    y = y * (jnp.float32(1.5) - hz * y * y)
    y = y * (jnp.float32(1.5) - hz * y * y)
    s = z * y
    return jnp.float32(0.5) * (s + z / s)


def _acosf(x):
    ax = jnp.abs(x)
    half = jnp.float32(0.5)
    one = jnp.float32(1.0)
    lo = ax < half
    neg = x < jnp.float32(0.0)
    z = jnp.where(lo, x * x, half * jnp.where(neg, one + x, one - x))
    p = z * (_PS0 + z * (_PS1 + z * _PS2))
    q = one + z * _QS1
    r = p / q
    s = _sqrtf(z)
    r1 = _PIO2_HI - (x - (_PIO2_LO - x * r))
    r2 = jnp.float32(2.0) * (_PIO2_HI - (s + (r * s - _PIO2_LO)))
    df = lax.bitcast_convert_type(
        lax.bitcast_convert_type(s, jnp.int32) & jnp.int32(~0xFFF),
        jnp.float32)
    c = (z - df * df) / (s + df)
    r3 = jnp.float32(2.0) * (df + (r * s + c))
    return jnp.where(lo, r1, jnp.where(neg, r2, r3))


def _make_sc_call(Bn, L, nseg, ncut):
    slab = (L // NW) * L
    shift = L.bit_length() - 1

    mesh = plsc.VectorSubcoreMesh(core_axis_name="c", subcore_axis_name="s",
                                  num_cores=NC, num_subcores=NS)

    @functools.partial(
        pl.kernel,
        out_type=jax.ShapeDtypeStruct((NW * LN,), jnp.float32),
        mesh=mesh,
        compiler_params=pltpu.CompilerParams(needs_layout_passes=False,
                                             use_tc_tiling_on_sc=False),
        scratch_types=[
            pltpu.VMEM((slab,), jnp.uint8),
            pltpu.VMEM((slab + 3 * CHUNK,), jnp.int32),
            pltpu.VMEM((6 * Bn * L,), jnp.float32),
            pltpu.VMEM((ncut * LN,), jnp.float32),
            pltpu.VMEM((8 * CHUNK,), jnp.int32),
            pltpu.VMEM((8 * CHUNK,), jnp.int32),
            pltpu.VMEM((2 * CHUNK,), jnp.float32),
            pltpu.VMEM((2 * CHUNK,), jnp.float32),
            pltpu.VMEM((2 * CHUNK,), jnp.float32),
            pltpu.VMEM((2 * CHUNK,), jnp.float32),
            pltpu.VMEM((8 * CHUNK,), jnp.float32),
            pltpu.VMEM((8 * CHUNK,), jnp.float32),
            pltpu.VMEM((LN,), jnp.float32),
            pltpu.SemaphoreType.DMA,
            pltpu.SemaphoreType.DMA,
        ],
    )
    def sc_call(mask_hbm, geom_hbm, cut_hbm, coeff_hbm, out_hbm,
                mslab, pairs, geomv, cutv, offs0, offs1,
                xls0, xls1, wts0, wts1, cbuf0, cbuf1, accv, sem0, sem1):
        bufs = ((offs0, xls0, wts0, cbuf0, sem0),
                (offs1, xls1, wts1, cbuf1, sem1))
        wid = lax.axis_index("s") * NC + lax.axis_index("c")
        base_pair = wid * slab

        pltpu.sync_copy(mask_hbm.at[pl.ds(base_pair, slab)], mslab)
        pltpu.sync_copy(geom_hbm, geomv)
        pltpu.sync_copy(cut_hbm, cutv)
        accv[...] = jnp.zeros((LN,), jnp.float32)

        iota = lax.iota(jnp.int32, LN)

        pairs_full = pairs.at[pl.ds(0, slab + 3 * CHUNK)]

        def compact_body(q, cnt):
            v8 = mslab[pl.ds(q * (4 * LN), 4 * LN)]
            x = plsc.bitcast(v8, jnp.int32)
            clane = lax.shift_right_logical(x * jnp.int32(0x01010101), 24)
            incl = plsc.cumsum(clane)
            base = cnt + incl - clane
            b0 = x & jnp.int32(0xFF)
            b1 = lax.shift_right_logical(x, 8) & jnp.int32(0xFF)
            b2 = lax.shift_right_logical(x, 16) & jnp.int32(0xFF)
            p1 = b0
            p2 = p1 + b1
            p3 = p2 + b2
            idbase = (base_pair + q * (4 * LN)) + iota * jnp.int32(4)
            for k, pk in ((0, None), (1, p1), (2, p2), (3, p3)):
                bk = lax.shift_right_logical(x, 8 * k) & jnp.int32(0xFF)
                mk = bk != 0
                posk = base if pk is None else base + pk
                plsc.store_scatter(pairs_full, [posk],
                                   idbase + jnp.int32(k), mask=mk)
            return cnt + incl[LN - 1]

        cnt = lax.fori_loop(0, slab // (4 * LN), compact_body, jnp.int32(0))

        nch2 = lax.shift_right_logical(cnt + jnp.int32(2 * CHUNK - 1), 8)

        def stage(g, bi):
            offs, xls, wts, cbuf, sem = bufs[bi]
            cbase = g * CHUNK
            for t in range(SUB):
                k0 = cbase + t * LN
                vm = (k0 + iota) < cnt
                p = jnp.where(vm, pairs[pl.ds(k0, LN)], 0)
                i = lax.shift_right_logical(p, shift)
                j = p & jnp.int32(L - 1)
                jblk = lax.shift_left(
                    lax.shift_right_logical(p, 7) & jnp.int32((L >> 7) - 1), 9)
                jin = p & jnp.int32(127)

                for b in range(Bn):
                    rb = 6 * b * L

                    def ld(row, idxv):
                        return plsc.load_gather(
                            geomv, [idxv + jnp.int32(rb + row * L)])

                    cbxi = ld(3, i)
                    cbyi = ld(4, i)
                    cbzi = ld(5, i)
                    xx = ld(0, i) - cbxi
                    xy = ld(1, i) - cbyi
                    xz = ld(2, i) - cbzi
                    yx = ld(3, j) - cbxi
                    yy = ld(4, j) - cbyi
                    yz = ld(5, j) - cbzi
                    nx2 = xx * xx + xy * xy + xz * xz
                    ny2 = yx * yx + yy * yy + yz * yz
                    dot = xx * yx + xy * yy + xz * yz
                    nx = _sqrtf(nx2)
                    ny = _sqrtf(ny2)
                    m1 = (nx > EPS) & (ny > EPS)
                    denom = jnp.where(m1, nx * ny, jnp.float32(1.0))
                    cth = dot / denom
                    good = m1 & ((jnp.float32(1.0) - cth * cth) > EPS) & vm
                    phi = _acosf(jnp.where(good, cth, jnp.float32(0.0)))

                    nlt = jnp.zeros((LN,), jnp.int32)
                    cutsel = cutv[pl.ds(0, LN)]
                    for k in range(ncut):
                        ck = cutv[pl.ds(k * LN, LN)]
                        m = phi > ck
                        nlt = nlt + jnp.where(m, jnp.int32(1), jnp.int32(0))
                        if 1 <= k <= nseg - 1:
                            cutsel = jnp.where(m, ck, cutsel)
                    idx = jnp.clip(nlt - jnp.int32(1), jnp.int32(0),
                                   jnp.int32(nseg - 1))
                    base = ((i * jnp.int32(nseg) + idx) * jnp.int32(4 * L)
                            + jblk + jin)
                    base = jnp.where(good, base, 0)

                    ob = 4 * b * CHUNK
                    offs[pl.ds(ob + t * LN, LN)] = base
                    offs[pl.ds(ob + CHUNK + t * LN, LN)] = base + jnp.int32(128)
                    offs[pl.ds(ob + 2 * CHUNK + t * LN, LN)] = (
                        base + jnp.int32(256))
                    offs[pl.ds(ob + 3 * CHUNK + t * LN, LN)] = (
                        base + jnp.int32(384))
                    xls[pl.ds(b * CHUNK + t * LN, LN)] = phi - cutsel
                    wts[pl.ds(b * CHUNK + t * LN, LN)] = jnp.where(
                        good, jnp.float32(1.0), jnp.float32(0.0))

            pltpu.async_copy(coeff_hbm.at[offs], cbuf, sem)

        def drain_eval(bi):
            offs, xls, wts, cbuf, sem = bufs[bi]
            pltpu.make_async_copy(coeff_hbm.at[offs], cbuf, sem).wait()
            for t in range(SUB):
                for b in range(Bn):
                    ob = 4 * b * CHUNK
                    c0 = cbuf[pl.ds(ob + t * LN, LN)]
                    c1 = cbuf[pl.ds(ob + CHUNK + t * LN, LN)]
                    c2 = cbuf[pl.ds(ob + 2 * CHUNK + t * LN, LN)]
                    c3 = cbuf[pl.ds(ob + 3 * CHUNK + t * LN, LN)]
                    xl = xls[pl.ds(b * CHUNK + t * LN, LN)]
                    w = wts[pl.ds(b * CHUNK + t * LN, LN)]
                    tx = xl * xl
                    ret = (c3 + c2 * xl) + c1 * tx
                    ret = ret + c0 * (tx * xl)
                    plsc.addupdate(accv.at[pl.ds(0, LN)], w * ret)

        stage(jnp.int32(0), 0)

        def pair_body(g2, carry):
            stage(g2 * 2 + 1, 1)
            drain_eval(0)
            stage(g2 * 2 + 2, 0)
            drain_eval(1)
            return carry

        lax.fori_loop(jnp.int32(0), nch2, pair_body, jnp.int32(0))
        drain_eval(0)

        pltpu.sync_copy(accv, out_hbm.at[pl.ds(wid * LN, LN)])

    return sc_call


def kernel(CA, CB, coeff, cutoffs, mask):
    Bn, L, _ = CA.shape
    nseg = coeff.shape[2]
    ncut = cutoffs.shape[0]
    geom = jnp.concatenate(
        [jnp.swapaxes(CA, 1, 2), jnp.swapaxes(CB, 1, 2)], axis=1)
    geom = geom.reshape(-1)
    mask_i = mask.reshape(-1).astype(jnp.uint8)
    coeff_flat = jnp.transpose(
        coeff.reshape(L, L // 128, 128, nseg, 4),
        (0, 3, 1, 4, 2)).reshape(-1)
    cut_bro = jnp.broadcast_to(cutoffs[:, None], (ncut, LN)).reshape(-1)
    sc_call = _make_sc_call(Bn, L, nseg, ncut)
    out = sc_call(mask_i, geom, cut_bro, coeff_flat)
    return jnp.sum(out)


# speedup vs baseline: 3.3679x; 1.0904x over previous
import functools

import numpy as np
import jax
import jax.numpy as jnp
from jax import lax
from jax.experimental import pallas as pl
from jax.experimental.pallas import tpu as pltpu
from jax.experimental.pallas import tpu_sc as plsc

NC = 2
NS = 16
LN = 16
NW = NC * NS
CHUNK = 128
SUB = CHUNK // LN
EPS = np.float32(1e-6)

_PIO2_HI = np.float32(1.5707962513e+00)
_PIO2_LO = np.float32(7.5497894159e-08)
_PS0 = np.float32(1.6666586697e-01)
_PS1 = np.float32(-4.2743422091e-02)
_PS2 = np.float32(-8.6563630030e-03)
_QS1 = np.float32(-7.0662963390e-01)
_MAGIC = np.int32(0x5F3759DF)


def _sqrtf(z):
    i = lax.bitcast_convert_type(z, jnp.int32)
    y = lax.bitcast_convert_type(_MAGIC - lax.shift_right_logical(i, 1),
                                 jnp.float32)
    hz = jnp.float32(0.5) * z
    y = y * (jnp.float32(1.5) - hz * y * y)
    y = y * (jnp.float32(1.5) - hz * y * y)
    s = z * y
    return jnp.float32(0.5) * (s + z / s)


def _acosf(x):
    ax = jnp.abs(x)
    half = jnp.float32(0.5)
    one = jnp.float32(1.0)
    lo = ax < half
    neg = x < jnp.float32(0.0)
    z = jnp.where(lo, x * x, half * jnp.where(neg, one + x, one - x))
    p = z * (_PS0 + z * (_PS1 + z * _PS2))
    q = one + z * _QS1
    r = p / q
    s = _sqrtf(z)
    r1 = _PIO2_HI - (x - (_PIO2_LO - x * r))
    r2 = jnp.float32(2.0) * (_PIO2_HI - (s + (r * s - _PIO2_LO)))
    df = lax.bitcast_convert_type(
        lax.bitcast_convert_type(s, jnp.int32) & jnp.int32(~0xFFF),
        jnp.float32)
    c = (z - df * df) / (s + df)
    r3 = jnp.float32(2.0) * (df + (r * s + c))
    return jnp.where(lo, r1, jnp.where(neg, r2, r3))


def _vgather(vec, idxv):
    dn = lax.GatherDimensionNumbers(offset_dims=(), collapsed_slice_dims=(0,),
                                    start_index_map=(0,))
    return lax.gather(vec, idxv.reshape(LN, 1), dn, (1,),
                      mode=lax.GatherScatterMode.PROMISE_IN_BOUNDS)


def _make_sc_call(Bn, L, nseg, ncut):
    slab = (L // NW) * L
    shift = L.bit_length() - 1

    mesh = plsc.VectorSubcoreMesh(core_axis_name="c", subcore_axis_name="s",
                                  num_cores=NC, num_subcores=NS)

    @functools.partial(
        pl.kernel,
        out_type=jax.ShapeDtypeStruct((NW * LN,), jnp.float32),
        mesh=mesh,
        compiler_params=pltpu.CompilerParams(needs_layout_passes=False,
                                             use_tc_tiling_on_sc=False),
        scratch_types=[
            pltpu.VMEM((slab,), jnp.uint8),
            pltpu.VMEM((slab + 3 * CHUNK,), jnp.int32),
            pltpu.VMEM((6 * Bn * L,), jnp.float32),
            pltpu.VMEM((ncut,), jnp.float32),
            pltpu.VMEM((4 * CHUNK,), jnp.int32),
            pltpu.VMEM((4 * CHUNK,), jnp.int32),
            pltpu.VMEM((CHUNK,), jnp.float32),
            pltpu.VMEM((CHUNK,), jnp.float32),
            pltpu.VMEM((CHUNK,), jnp.float32),
            pltpu.VMEM((CHUNK,), jnp.float32),
            pltpu.VMEM((4 * CHUNK,), jnp.float32),
            pltpu.VMEM((4 * CHUNK,), jnp.float32),
            pltpu.VMEM((LN,), jnp.float32),
            pltpu.SemaphoreType.DMA,
            pltpu.SemaphoreType.DMA,
        ],
    )
    def sc_call(mask_hbm, geom_hbm, cut_hbm, coeff_hbm, out_hbm,
                mslab, pairs, geomv, cutv, offs0, offs1,
                xls0, xls1, wts0, wts1, cbuf0, cbuf1, accv, sem0, sem1):
        bufs = ((offs0, xls0, wts0, cbuf0, sem0),
                (offs1, xls1, wts1, cbuf1, sem1))
        wid = lax.axis_index("s") * NC + lax.axis_index("c")
        base_pair = wid * slab

        pltpu.sync_copy(mask_hbm.at[pl.ds(base_pair, slab)], mslab)
        pltpu.sync_copy(geom_hbm, geomv)
        pltpu.sync_copy(cut_hbm, cutv)
        accv[...] = jnp.zeros((LN,), jnp.float32)

        iota = lax.iota(jnp.int32, LN)

        pairs_full = pairs.at[pl.ds(0, slab + 3 * CHUNK)]

        def compact_body(q, cnt):
            v8 = mslab[pl.ds(q * (4 * LN), 4 * LN)]
            x = plsc.bitcast(v8, jnp.int32)
            clane = lax.shift_right_logical(x * jnp.int32(0x01010101), 24)
            incl = plsc.cumsum(clane)
            base = cnt + incl - clane
            b0 = x & jnp.int32(0xFF)
            b1 = lax.shift_right_logical(x, 8) & jnp.int32(0xFF)
            b2 = lax.shift_right_logical(x, 16) & jnp.int32(0xFF)
            p1 = b0
            p2 = p1 + b1
            p3 = p2 + b2
            idbase = (base_pair + q * (4 * LN)) + iota * jnp.int32(4)
            for k, pk in ((0, None), (1, p1), (2, p2), (3, p3)):
                bk = lax.shift_right_logical(x, 8 * k) & jnp.int32(0xFF)
                mk = bk != 0
                posk = base if pk is None else base + pk
                plsc.store_scatter(pairs_full, [posk],
                                   idbase + jnp.int32(k), mask=mk)
            return cnt + incl[LN - 1]

        cnt = lax.fori_loop(0, slab // (4 * LN), compact_body, jnp.int32(0))

        nch2 = lax.shift_right_logical(cnt + jnp.int32(2 * CHUNK - 1), 8)

        def make_stage(b):
            rb = 6 * b * L

            def stage(g, bi):
                offs, xls, wts, cbuf, sem = bufs[bi]
                cbase = g * CHUNK
                for t in range(SUB):
                    k0 = cbase + t * LN
                    vm = (k0 + iota) < cnt
                    p = jnp.where(vm, pairs[pl.ds(k0, LN)], 0)
                    i = lax.shift_right_logical(p, shift)
                    j = p & jnp.int32(L - 1)

                    def ld(row, idxv):
                        return plsc.load_gather(geomv,
                                                [idxv + jnp.int32(rb + row * L)])

                    cbxi = ld(3, i)
                    cbyi = ld(4, i)
                    cbzi = ld(5, i)
                    xx = ld(0, i) - cbxi
                    xy = ld(1, i) - cbyi
                    xz = ld(2, i) - cbzi
                    yx = ld(3, j) - cbxi
                    yy = ld(4, j) - cbyi
                    yz = ld(5, j) - cbzi
                    nx2 = xx * xx + xy * xy + xz * xz
                    ny2 = yx * yx + yy * yy + yz * yz
                    dot = xx * yx + xy * yy + xz * yz
                    nx = _sqrtf(nx2)
                    ny = _sqrtf(ny2)
                    m1 = (nx > EPS) & (ny > EPS)
                    denom = jnp.where(m1, nx * ny, jnp.float32(1.0))
                    cth = dot / denom
                    good = m1 & ((jnp.float32(1.0) - cth * cth) > EPS) & vm
                    phi = _acosf(jnp.where(good, cth, jnp.float32(0.0)))

                    cutvec = cutv[pl.ds(0, ncut)]
                    cnt16 = jnp.zeros((LN,), jnp.int32)
                    for step in (8, 4, 2, 1):
                        cv = _vgather(cutvec, cnt16 + jnp.int32(step - 1))
                        cnt16 = cnt16 + jnp.where(cv < phi, jnp.int32(step),
                                                  jnp.int32(0))
                    idx = jnp.clip(cnt16 - jnp.int32(1), jnp.int32(0),
                                   jnp.int32(nseg - 1))
                    cutsel = _vgather(cutvec, idx)
                    base = ((i * jnp.int32(nseg) + idx) * jnp.int32(4 * L)
                            + lax.shift_left(lax.shift_right_logical(p, 7)
                                             & jnp.int32((L >> 7) - 1), 9)
                            + (p & jnp.int32(127)))
                    base = jnp.where(good, base, 0)

                    offs[pl.ds(t * LN, LN)] = base
                    offs[pl.ds(CHUNK + t * LN, LN)] = base + jnp.int32(128)
                    offs[pl.ds(2 * CHUNK + t * LN, LN)] = base + jnp.int32(256)
                    offs[pl.ds(3 * CHUNK + t * LN, LN)] = base + jnp.int32(384)
                    xls[pl.ds(t * LN, LN)] = phi - cutsel
                    wts[pl.ds(t * LN, LN)] = jnp.where(good, jnp.float32(1.0),
                                                       jnp.float32(0.0))

                pltpu.async_copy(coeff_hbm.at[offs], cbuf, sem)

            return stage

        def drain_eval(bi):
            offs, xls, wts, cbuf, sem = bufs[bi]
            pltpu.make_async_copy(coeff_hbm.at[offs], cbuf, sem).wait()
            for t in range(SUB):
                c0 = cbuf[pl.ds(t * LN, LN)]
                c1 = cbuf[pl.ds(CHUNK + t * LN, LN)]
                c2 = cbuf[pl.ds(2 * CHUNK + t * LN, LN)]
                c3 = cbuf[pl.ds(3 * CHUNK + t * LN, LN)]
                xl = xls[pl.ds(t * LN, LN)]
                w = wts[pl.ds(t * LN, LN)]
                tx = xl * xl
                ret = (c3 + c2 * xl) + c1 * tx
                ret = ret + c0 * (tx * xl)
                plsc.addupdate(accv.at[pl.ds(0, LN)], w * ret)

        for b in range(Bn):
            stage = make_stage(b)
            stage(jnp.int32(0), 0)

            def pair_body(g2, carry):
                stage(g2 * 2 + 1, 1)
                drain_eval(0)
                stage(g2 * 2 + 2, 0)
                drain_eval(1)
                return carry

            lax.fori_loop(jnp.int32(0), nch2, pair_body, jnp.int32(0))
            drain_eval(0)

        pltpu.sync_copy(accv, out_hbm.at[pl.ds(wid * LN, LN)])

    return sc_call


def kernel(CA, CB, coeff, cutoffs, mask):
    Bn, L, _ = CA.shape
    nseg = coeff.shape[2]
    ncut = cutoffs.shape[0]
    geom = jnp.concatenate(
        [jnp.swapaxes(CA, 1, 2), jnp.swapaxes(CB, 1, 2)], axis=1)
    geom = geom.reshape(-1)
    mask_i = mask.reshape(-1).astype(jnp.uint8)
    coeff_flat = jnp.transpose(
        coeff.reshape(L, L // 128, 128, nseg, 4),
        (0, 3, 1, 4, 2)).reshape(-1)
    cut_bro = cutoffs
    sc_call = _make_sc_call(Bn, L, nseg, ncut)
    out = sc_call(mask_i, geom, cut_bro, coeff_flat)
    return jnp.sum(out)
